# baseline (speedup 1.0000x reference)
import functools

import jax
import jax.numpy as jnp
from jax import lax
from jax.experimental import pallas as pl
from jax.experimental.pallas import tpu as pltpu
from jax.experimental.pallas import tpu_sc as plsc

EMBED = 64
NC = 2
NS = 16
NW = NC * NS
BB = 128
NBUF = 4
SBUF = 2


@jax.jit
def _sc_embed(seqT, table2):
    S, B = seqT.shape
    nb = B // BB
    nch = S
    mesh = plsc.VectorSubcoreMesh(core_axis_name="c", subcore_axis_name="s")

    @functools.partial(
        pl.kernel,
        mesh=mesh,
        out_type=jax.ShapeDtypeStruct((S, EMBED // 8, nb, 8, BB), jnp.float32),
        scratch_types=[
            pltpu.VMEM((S, BB), jnp.int32),
            pltpu.VMEM((NBUF, BB), jnp.int32),
            pltpu.VMEM((NBUF, BB, 2 * EMBED), jnp.float32),
            pltpu.VMEM((SBUF, EMBED // 8, 8, BB), jnp.float32),
            pltpu.SemaphoreType.DMA,
            pltpu.SemaphoreType.DMA,
        ],
        compiler_params=pltpu.CompilerParams(
            use_tc_tiling_on_sc=False, needs_layout_passes=False
        ),
    )
    def k(seq_hbm, tab2_hbm, out_hbm, idx_v, idx2_v, rows_v, tbuf_v, gsem, ssem):
        wid = lax.axis_index("s") * NC + lax.axis_index("c")
        pltpu.sync_copy(seq_hbm.at[:, pl.ds(wid * BB, BB)], idx_v)

        lanes = lax.iota(jnp.int32, 16)

        def fill_pair_indices(i, b):
            for g in range(BB // 16):
                idx2_v[b, pl.ds(16 * g, 16)] = (
                    idx_v[i, pl.ds(16 * g, 16)] >> 1
                )

        def start_gather(b):
            pltpu.async_copy(tab2_hbm.at[idx2_v.at[b]], rows_v.at[b], gsem)

        def wait_gather(b):
            pltpu.make_async_copy(
                tab2_hbm.at[idx2_v.at[b]], rows_v.at[b], gsem
            ).wait()

        def start_store(i, b):
            pltpu.async_copy(tbuf_v.at[b], out_hbm.at[i, :, wid], ssem)

        def wait_store(i, b):
            pltpu.make_async_copy(
                tbuf_v.at[b], out_hbm.at[i, :, wid], ssem
            ).wait()

        def transpose(i, b):
            rows = rows_v.at[b]
            tb = tbuf_v.at[b % SBUF]

            @plsc.parallel_loop(0, BB // 16, step=1, unroll=2)
            def tr(jg):
                jrow = lanes + 16 * jg
                pvec = (idx_v[i, pl.ds(16 * jg, 16)] & 1) * EMBED

                @plsc.parallel_loop(0, EMBED // 8, step=1, unroll=4)
                def tr2(ti):
                    for r in range(8):
                        v = plsc.load_gather(rows, [jrow, pvec + (ti * 8 + r)])
                        tb[ti, r, pl.ds(16 * jg, 16)] = v

        for b in range(NBUF):
            fill_pair_indices(b, b)
            start_gather(b)

        for b in range(NBUF):
            wait_gather(b)
            if b >= SBUF:
                wait_store(b - SBUF, b % SBUF)
            transpose(b, b)
            start_store(b, b % SBUF)
            fill_pair_indices(b + NBUF, b)
            start_gather(b)

        def group(g, carry):
            for b in range(NBUF):
                i = g * NBUF + b
                wait_gather(b)
                wait_store(i - SBUF, b % SBUF)
                transpose(i, b)
                start_store(i, b % SBUF)
                fill_pair_indices(i + NBUF, b)
                start_gather(b)
            return carry

        lax.fori_loop(1, nch // NBUF - 1, group, 0)

        for b in range(NBUF):
            i = nch - NBUF + b
            wait_gather(b)
            wait_store(i - SBUF, b % SBUF)
            transpose(i, b)
            start_store(i, b % SBUF)

        for b in range(SBUF):
            i = nch - SBUF + b
            wait_store(i, i % SBUF)

    return k(seqT, table2)


def kernel(sequence, table):
    B, S = sequence.shape
    seqT = sequence.T.astype(jnp.int32)
    table2 = table.astype(jnp.float32).reshape(-1, 2 * EMBED)
    out6 = _sc_embed(seqT, table2)
    return out6.transpose(2, 4, 0, 1, 3).reshape(B, S, EMBED)


# speedup vs baseline: 1.0685x; 1.0685x over previous
import functools

import jax
import jax.numpy as jnp
from jax import lax
from jax.experimental import pallas as pl
from jax.experimental.pallas import tpu as pltpu
from jax.experimental.pallas import tpu_sc as plsc

EMBED = 64
NC = 2
NS = 16
NW = NC * NS
CH = 128
NBUF = 4


@functools.partial(jax.jit, static_argnames=("nch",))
def _sc_gather(seq3, table, nch):
    bpw = nch * CH
    mesh = plsc.VectorSubcoreMesh(core_axis_name="c", subcore_axis_name="s")

    @functools.partial(
        pl.kernel,
        mesh=mesh,
        out_type=jax.ShapeDtypeStruct((NW * bpw, EMBED), jnp.float32),
        scratch_types=[
            pltpu.VMEM((nch, CH), jnp.int32),
            pltpu.VMEM((NBUF, CH, EMBED), jnp.float32),
            pltpu.SemaphoreType.DMA,
        ],
        compiler_params=pltpu.CompilerParams(use_tc_tiling_on_sc=False),
    )
    def k(seq_hbm, tab_hbm, out_hbm, idx_v, rows_v, gsem):
        wid = lax.axis_index("s") * NC + lax.axis_index("c")
        base = wid * bpw
        pltpu.sync_copy(seq_hbm.at[wid], idx_v)

        for b in range(NBUF):
            pltpu.async_copy(tab_hbm.at[idx_v.at[b]], rows_v.at[b], gsem)

        def group(g, carry):
            for b in range(NBUF):
                i = g * NBUF + b
                pltpu.make_async_copy(
                    tab_hbm.at[idx_v.at[i]], rows_v.at[b], gsem
                ).wait()
                pltpu.sync_copy(
                    rows_v.at[b], out_hbm.at[pl.ds(base + i * CH, CH)]
                )
                pltpu.async_copy(
                    tab_hbm.at[idx_v.at[i + NBUF]], rows_v.at[b], gsem
                )
            return carry

        lax.fori_loop(0, nch // NBUF - 1, group, 0)

        for b in range(NBUF):
            i = nch - NBUF + b
            pltpu.make_async_copy(
                tab_hbm.at[idx_v.at[i]], rows_v.at[b], gsem
            ).wait()
            pltpu.sync_copy(rows_v.at[b], out_hbm.at[pl.ds(base + i * CH, CH)])

    return k(seq3, table)


def kernel(sequence, table):
    n_tok = sequence.size
    assert n_tok % (NW * CH * NBUF) == 0
    nch = n_tok // (NW * CH)
    seq3 = sequence.reshape(NW, nch, CH).astype(jnp.int32)
    out = _sc_gather(seq3, table.astype(jnp.float32), nch)
    return out.reshape(sequence.shape + (EMBED,))
